# baseline (speedup 1.0000x reference)
import functools

import jax
import jax.numpy as jnp
from jax import lax
from jax.experimental import pallas as pl
from jax.experimental.pallas import tpu as pltpu
from jax.experimental.pallas import tpu_sc as plsc

B = 16384
D = 16
NC = 2
NS = 16
NW = NC * NS
CHUNK = B // NW
SUB = 128
NSUB = CHUNK // SUB


def _body(uidx_h, iidx_h, umf_h, imf_h, umlp_h, imlp_h, umf2_h, wts_h, out_h,
          uidx_v, iidx_v, umf_v, imf_v, umlp_v, imlp_v,
          comb_v, out_v, wts_v, sem0, sem1, sem2, sem3):
  wid = lax.axis_index("s") * NC + lax.axis_index("c")
  base = pl.multiple_of(wid * CHUNK, CHUNK)
  pltpu.sync_copy(wts_h, wts_v)
  pltpu.sync_copy(uidx_h.at[pl.ds(base, CHUNK)], uidx_v)
  pltpu.sync_copy(iidx_h.at[pl.ds(base, CHUNK)], iidx_v)

  wmf = wts_v[0]
  vu = wts_v[1]
  vi = wts_v[2]
  c0v = wts_v[3]
  lanes = lax.iota(jnp.int32, 16)
  rows16 = lanes * D

  for t in range(NSUB):
    t0 = t * SUB

    def fire(g, _):
      r0 = pl.multiple_of(t0 + g * D, D)
      u_vec = uidx_v[pl.ds(r0, D)]
      i_vec = iidx_v[pl.ds(r0, D)]
      for j in range(D):
        u = u_vec[j]
        i = i_vec[j]
        row = g * D + j
        pltpu.async_copy(umf_h.at[u], umf_v.at[row], sem0)
        pltpu.async_copy(imf_h.at[i], imf_v.at[row], sem1)
        pltpu.async_copy(umlp_h.at[u], umlp_v.at[row], sem2)
        pltpu.async_copy(imlp_h.at[i], imlp_v.at[row], sem3)
      return 0

    lax.fori_loop(0, SUB // D, fire, 0)

    dummy = umf2_h
    pltpu.make_async_copy(dummy, umf_v, sem0).wait()
    pltpu.make_async_copy(dummy, imf_v, sem1).wait()
    pltpu.make_async_copy(dummy, umlp_v, sem2).wait()
    pltpu.make_async_copy(dummy, imlp_v, sem3).wait()

    def grp(g, carry):
      wmf, vu, vi, c0v, rows16 = carry
      r0 = pl.multiple_of(g * D, D)
      for j in range(D):
        comb_v[pl.ds(j * D, D)] = (umf_v[r0 + j] * imf_v[r0 + j] * wmf
                                   + umlp_v[r0 + j] * vu
                                   + imlp_v[r0 + j] * vi)
      acc = c0v
      for d in range(D):
        acc = acc + plsc.load_gather(comb_v, [rows16 + d])
      out_v[pl.ds(t0 + r0, D)] = acc
      return carry

    lax.fori_loop(0, SUB // D, grp, (wmf, vu, vi, c0v, rows16))

  pltpu.sync_copy(out_v, out_h.at[pl.ds(base, CHUNK)])


@jax.jit
def _run(uidx, iidx, umf, imf, umlp, imlp, umf2, wts):
  mesh = plsc.VectorSubcoreMesh(core_axis_name="c", subcore_axis_name="s",
                                num_cores=NC, num_subcores=NS)
  f = functools.partial(
      pl.kernel,
      out_type=jax.ShapeDtypeStruct((B,), jnp.float32),
      mesh=mesh,
      compiler_params=pltpu.CompilerParams(needs_layout_passes=False),
      scratch_types=[
          pltpu.VMEM((CHUNK,), jnp.int32),
          pltpu.VMEM((CHUNK,), jnp.int32),
          pltpu.VMEM((SUB, D), jnp.float32),
          pltpu.VMEM((SUB, D), jnp.float32),
          pltpu.VMEM((SUB, D), jnp.float32),
          pltpu.VMEM((SUB, D), jnp.float32),
          pltpu.VMEM((D * D,), jnp.float32),
          pltpu.VMEM((CHUNK,), jnp.float32),
          pltpu.VMEM((4, D), jnp.float32),
          pltpu.SemaphoreType.DMA,
          pltpu.SemaphoreType.DMA,
          pltpu.SemaphoreType.DMA,
          pltpu.SemaphoreType.DMA,
      ])(_body)
  return f(uidx, iidx, umf, imf, umlp, imlp, umf2, wts)


def kernel(users_index, items_index, user_mf, item_mf, user_mlp, item_mlp,
           W1, b1, Wp, bp):
  wmf = Wp[:D, 0]
  wp2 = Wp[D:, 0]
  v = W1 @ wp2
  c0 = b1 @ wp2 + bp[0]
  wts = jnp.stack([wmf, v[:D], v[D:], jnp.broadcast_to(c0, (D,))])
  return _run(users_index.astype(jnp.int32), items_index.astype(jnp.int32),
              user_mf, item_mf, user_mlp, item_mlp,
              jnp.zeros((SUB, D), jnp.float32), wts)
